# baseline (speedup 1.0000x reference)
import functools

import jax
import jax.numpy as jnp
from jax import lax
from jax.experimental import pallas as pl
from jax.experimental.pallas import tpu as pltpu
from jax.experimental.pallas import tpu_sc as plsc

_TOTAL_BITS = 2048
_N = 8192
_NBITS = 12
_B = 256
_ROW = 1 << _NBITS

_NB = 512
_GRID = _N // _NB

_NC = 2
_NS = 16
_NW = _NC * _NS
_TOTAL = _B * _N
_PER_W = _TOTAL // _NW
_LW = 128
_TPS = 2
_K = 8 * _TPS
_NCH = 64 // _TPS


_W8 = _TOTAL_BITS // 8


def _addr_body(bits_ref, conn_ref, idx_ref, pack_ref):
    blk = pl.program_id(0)

    @pl.when(blk == 0)
    def _():
        bits = bits_ref[...].astype(jnp.bfloat16)
        k_iota = lax.broadcasted_iota(jnp.int32, (_TOTAL_BITS, _W8), 0)
        g_iota = lax.broadcasted_iota(jnp.int32, (_TOTAL_BITS, _W8), 1)
        pows = jnp.left_shift(jnp.int32(1), k_iota & 7)
        P = jnp.where((k_iota >> 3) == g_iota, pows, 0).astype(jnp.bfloat16)
        pack_ref[...] = jnp.dot(
            bits, P, preferred_element_type=jnp.float32
        ).astype(jnp.bfloat16)

    pack = pack_ref[...]
    conn = conn_ref[...]
    word_id = conn >> 3
    bit_pos = conn & 7

    addr = jnp.zeros((_B, _NB), jnp.int32)
    for j in range(_NBITS):
        gj = word_id[:, j]
        u_iota = lax.broadcasted_iota(jnp.int32, (_W8, _NB), 0)
        onehot = (u_iota == gj[None, :]).astype(jnp.bfloat16)
        A = jnp.dot(pack, onehot, preferred_element_type=jnp.float32)
        word = A.astype(jnp.int32)
        bit = (word >> bit_pos[:, j][None, :]) & 1
        addr += bit << j

    base = blk * _NB
    neuron = base + lax.broadcasted_iota(jnp.int32, (_B, _NB), 1)
    idx_ref[...] = (
        ((neuron >> 3) << 15)
        + ((addr >> 7) << 10)
        + ((neuron & 7) << 7)
        + (addr & 127)
    )


_addr_call = pl.pallas_call(
    _addr_body,
    grid=(_GRID,),
    in_specs=[
        pl.BlockSpec((_B, _TOTAL_BITS), lambda i: (0, 0)),
        pl.BlockSpec((_NB, _NBITS), lambda i: (i, 0)),
    ],
    out_specs=pl.BlockSpec((_B, _NB), lambda i: (0, i)),
    out_shape=jax.ShapeDtypeStruct((_B, _N), jnp.int32),
    scratch_shapes=[pltpu.VMEM((_B, _W8), jnp.bfloat16)],
)


def _gather_body(mem_hbm, idx_hbm, out_hbm, idx_v, val_v, sem):
    wid = lax.axis_index("s") * _NC + lax.axis_index("c")
    band = wid * 8

    def chunk(i, _):
        col = i * _TPS * _LW
        for t in range(_TPS):
            pltpu.sync_copy(
                idx_hbm.at[pl.ds(band, 8), pl.ds(col + t * _LW, _LW)],
                idx_v.at[pl.ds(t * 8, 8)],
            )
        handles = [
            pltpu.async_copy(mem_hbm.at[idx_v.at[j]], val_v.at[j], sem)
            for j in range(_K)
        ]
        for h in handles:
            h.wait()
        for t in range(_TPS):
            pltpu.sync_copy(
                val_v.at[pl.ds(t * 8, 8)],
                out_hbm.at[pl.ds(band, 8), pl.ds(col + t * _LW, _LW)],
            )
        return 0

    lax.fori_loop(0, _NCH, chunk, 0)


@functools.cache
def _gather_call():
    return functools.partial(
        pl.kernel,
        out_type=jax.ShapeDtypeStruct((_B, _N), jnp.float32),
        compiler_params=pltpu.CompilerParams(use_tc_tiling_on_sc=True),
        mesh=plsc.VectorSubcoreMesh(
            core_axis_name="c", subcore_axis_name="s",
            num_cores=_NC, num_subcores=_NS,
        ),
        scratch_types=[
            pltpu.VMEM((_K, _LW), jnp.int32),
            pltpu.VMEM((_K, _LW), jnp.float32),
            pltpu.SemaphoreType.DMA,
        ],
    )(_gather_body)


def kernel(input_bits, connections, memory):
    flat_idx = _addr_call(input_bits, connections)
    mem_lin = memory.reshape(1024, 8, 32, 128).transpose(0, 2, 1, 3).reshape(-1)
    return _gather_call()(mem_lin, flat_idx)


# speedup vs baseline: 1.0582x; 1.0582x over previous
import functools

import jax
import jax.numpy as jnp
from jax import lax
from jax.experimental import pallas as pl
from jax.experimental.pallas import tpu as pltpu
from jax.experimental.pallas import tpu_sc as plsc

_TOTAL_BITS = 2048
_N = 8192
_NBITS = 12
_B = 256
_ROW = 1 << _NBITS

_NB = 512
_GRID = _N // _NB

_NC = 2
_NS = 16
_NW = _NC * _NS
_TOTAL = _B * _N
_PER_W = _TOTAL // _NW
_LW = 128
_TPS = 4
_K = 8 * _TPS
_NCH = 64 // _TPS


_W8 = _TOTAL_BITS // 8


def _addr_body(bits_ref, conn_ref, idx_ref, pack_ref):
    blk = pl.program_id(0)

    @pl.when(blk == 0)
    def _():
        bits = bits_ref[...].astype(jnp.bfloat16)
        k_iota = lax.broadcasted_iota(jnp.int32, (_TOTAL_BITS, _W8), 0)
        g_iota = lax.broadcasted_iota(jnp.int32, (_TOTAL_BITS, _W8), 1)
        pows = jnp.left_shift(jnp.int32(1), k_iota & 7)
        P = jnp.where((k_iota >> 3) == g_iota, pows, 0).astype(jnp.bfloat16)
        pack_ref[...] = jnp.dot(
            bits, P, preferred_element_type=jnp.float32
        ).astype(jnp.bfloat16)

    pack = pack_ref[...]
    conn = conn_ref[...]
    word_id = conn >> 3
    bit_pos = conn & 7

    addr = jnp.zeros((_B, _NB), jnp.int32)
    for j in range(_NBITS):
        gj = word_id[:, j]
        u_iota = lax.broadcasted_iota(jnp.int32, (_W8, _NB), 0)
        onehot = (u_iota == gj[None, :]).astype(jnp.bfloat16)
        A = jnp.dot(pack, onehot, preferred_element_type=jnp.float32)
        word = A.astype(jnp.int32)
        bit = (word >> bit_pos[:, j][None, :]) & 1
        addr += bit << j

    base = blk * _NB
    neuron = base + lax.broadcasted_iota(jnp.int32, (_B, _NB), 1)
    idx_ref[...] = (
        ((neuron >> 3) << 15)
        + ((addr >> 7) << 10)
        + ((neuron & 7) << 7)
        + (addr & 127)
    )


_addr_call = pl.pallas_call(
    _addr_body,
    grid=(_GRID,),
    in_specs=[
        pl.BlockSpec((_B, _TOTAL_BITS), lambda i: (0, 0)),
        pl.BlockSpec((_NB, _NBITS), lambda i: (i, 0)),
    ],
    out_specs=pl.BlockSpec((_B, _NB), lambda i: (0, i)),
    out_shape=jax.ShapeDtypeStruct((_B, _N), jnp.int32),
    scratch_shapes=[pltpu.VMEM((_B, _W8), jnp.bfloat16)],
)


def _gather_body(mem_hbm, idx_hbm, out_hbm, idx_v, val_v, sem):
    wid = lax.axis_index("s") * _NC + lax.axis_index("c")
    band = wid * 8

    def chunk(i, _):
        col = i * _TPS * _LW
        for t in range(_TPS):
            pltpu.sync_copy(
                idx_hbm.at[pl.ds(band, 8), pl.ds(col + t * _LW, _LW)],
                idx_v.at[pl.ds(t * 8, 8)],
            )
        handles = [
            pltpu.async_copy(mem_hbm.at[idx_v.at[j]], val_v.at[j], sem)
            for j in range(_K)
        ]
        for h in handles:
            h.wait()
        for t in range(_TPS):
            pltpu.sync_copy(
                val_v.at[pl.ds(t * 8, 8)],
                out_hbm.at[pl.ds(band, 8), pl.ds(col + t * _LW, _LW)],
            )
        return 0

    lax.fori_loop(0, _NCH, chunk, 0)


@functools.cache
def _gather_call():
    return functools.partial(
        pl.kernel,
        out_type=jax.ShapeDtypeStruct((_B, _N), jnp.float32),
        compiler_params=pltpu.CompilerParams(use_tc_tiling_on_sc=True),
        mesh=plsc.VectorSubcoreMesh(
            core_axis_name="c", subcore_axis_name="s",
            num_cores=_NC, num_subcores=_NS,
        ),
        scratch_types=[
            pltpu.VMEM((_K, _LW), jnp.int32),
            pltpu.VMEM((_K, _LW), jnp.float32),
            pltpu.SemaphoreType.DMA,
        ],
    )(_gather_body)


def kernel(input_bits, connections, memory):
    flat_idx = _addr_call(input_bits, connections)
    mem_lin = memory.reshape(1024, 8, 32, 128).transpose(0, 2, 1, 3).reshape(-1)
    return _gather_call()(mem_lin, flat_idx)


# speedup vs baseline: 1.3475x; 1.2734x over previous
import functools

import jax
import jax.numpy as jnp
from jax import lax
from jax.experimental import pallas as pl
from jax.experimental.pallas import tpu as pltpu
from jax.experimental.pallas import tpu_sc as plsc

_TOTAL_BITS = 2048
_N = 8192
_NBITS = 12
_B = 256
_ROW = 1 << _NBITS

_NB = 512
_GRID = _N // _NB

_NC = 2
_NS = 16
_NW = _NC * _NS
_TOTAL = _B * _N
_PER_W = _TOTAL // _NW
_LW = 128
_TPS = 4
_K = 8 * _TPS
_NCH = 64 // _TPS


_W8 = _TOTAL_BITS // 8


def _addr_body(bits_ref, conn_ref, idx_ref, pack_ref):
    blk = pl.program_id(0)

    @pl.when(blk == 0)
    def _():
        bits = bits_ref[...].astype(jnp.bfloat16)
        k_iota = lax.broadcasted_iota(jnp.int32, (_TOTAL_BITS, _W8), 0)
        g_iota = lax.broadcasted_iota(jnp.int32, (_TOTAL_BITS, _W8), 1)
        pows = jnp.left_shift(jnp.int32(1), k_iota & 7)
        P = jnp.where((k_iota >> 3) == g_iota, pows, 0).astype(jnp.bfloat16)
        pack_ref[...] = jnp.dot(
            bits, P, preferred_element_type=jnp.float32
        ).astype(jnp.bfloat16)

    pack = pack_ref[...]
    conn = conn_ref[...]
    word_id = conn >> 3
    bit_pos = conn & 7

    addr = jnp.zeros((_B, _NB), jnp.int32)
    for j in range(_NBITS):
        gj = word_id[:, j]
        u_iota = lax.broadcasted_iota(jnp.int32, (_W8, _NB), 0)
        onehot = (u_iota == gj[None, :]).astype(jnp.bfloat16)
        A = jnp.dot(pack, onehot, preferred_element_type=jnp.float32)
        word = A.astype(jnp.int32)
        bit = (word >> bit_pos[:, j][None, :]) & 1
        addr += bit << j

    base = blk * _NB
    neuron = base + lax.broadcasted_iota(jnp.int32, (_B, _NB), 1)
    idx_ref[...] = (
        ((neuron >> 3) << 15)
        + ((addr >> 7) << 10)
        + ((neuron & 7) << 7)
        + (addr & 127)
    )


_addr_call = pl.pallas_call(
    _addr_body,
    grid=(_GRID,),
    in_specs=[
        pl.BlockSpec((_B, _TOTAL_BITS), lambda i: (0, 0)),
        pl.BlockSpec((_NB, _NBITS), lambda i: (i, 0)),
    ],
    out_specs=pl.BlockSpec((_B, _NB), lambda i: (0, i)),
    out_shape=jax.ShapeDtypeStruct((_B, _N), jnp.int32),
    scratch_shapes=[pltpu.VMEM((_B, _W8), jnp.bfloat16)],
)


def _gather_body(mem_hbm, idx_hbm, out_hbm, idx_v, val_v, isem, gsem, osem):
    wid = lax.axis_index("s") * _NC + lax.axis_index("c")
    band = wid * 8

    def idx_src(i, t):
        col = i * _TPS * _LW
        return idx_hbm.at[pl.ds(band, 8), pl.ds(col + t * _LW, _LW)]

    def out_dst(i, t):
        col = i * _TPS * _LW
        return out_hbm.at[pl.ds(band, 8), pl.ds(col + t * _LW, _LW)]

    for t in range(_TPS):
        pltpu.async_copy(idx_src(0, t), idx_v.at[0].at[pl.ds(t * 8, 8)], isem)

    def step(i, b):
        for t in range(_TPS):
            pltpu.make_async_copy(
                idx_src(i, t), idx_v.at[b].at[pl.ds(t * 8, 8)], isem
            ).wait()

        @pl.when(i >= 2)
        def _():
            for t in range(_TPS):
                pltpu.make_async_copy(
                    val_v.at[b].at[pl.ds(t * 8, 8)], out_dst(i - 2, t), osem
                ).wait()

        for j in range(_K):
            pltpu.async_copy(mem_hbm.at[idx_v.at[b].at[j]], val_v.at[b].at[j], gsem)

        @pl.when(i + 1 < _NCH)
        def _():
            for t in range(_TPS):
                pltpu.async_copy(
                    idx_src(i + 1, t), idx_v.at[1 - b].at[pl.ds(t * 8, 8)], isem
                )

        for j in range(_K):
            pltpu.make_async_copy(
                mem_hbm.at[idx_v.at[b].at[j]], val_v.at[b].at[j], gsem
            ).wait()

        for t in range(_TPS):
            pltpu.async_copy(val_v.at[b].at[pl.ds(t * 8, 8)], out_dst(i, t), osem)

    def pair(k, c):
        step(2 * k, 0)
        step(2 * k + 1, 1)
        return c

    lax.fori_loop(0, _NCH // 2, pair, 0)

    for i in (_NCH - 2, _NCH - 1):
        for t in range(_TPS):
            pltpu.make_async_copy(
                val_v.at[i % 2].at[pl.ds(t * 8, 8)], out_dst(i, t), osem
            ).wait()


@functools.cache
def _gather_call():
    return functools.partial(
        pl.kernel,
        out_type=jax.ShapeDtypeStruct((_B, _N), jnp.float32),
        compiler_params=pltpu.CompilerParams(use_tc_tiling_on_sc=True),
        mesh=plsc.VectorSubcoreMesh(
            core_axis_name="c", subcore_axis_name="s",
            num_cores=_NC, num_subcores=_NS,
        ),
        scratch_types=[
            pltpu.VMEM((2, _K, _LW), jnp.int32),
            pltpu.VMEM((2, _K, _LW), jnp.float32),
            pltpu.SemaphoreType.DMA,
            pltpu.SemaphoreType.DMA,
            pltpu.SemaphoreType.DMA,
        ],
    )(_gather_body)


def kernel(input_bits, connections, memory):
    flat_idx = _addr_call(input_bits, connections)
    mem_lin = memory.reshape(1024, 8, 32, 128).transpose(0, 2, 1, 3).reshape(-1)
    return _gather_call()(mem_lin, flat_idx)
